# baseline (speedup 1.0000x reference)
import functools

import jax
import jax.numpy as jnp
from jax.experimental import pallas as pl
from jax.experimental.pallas import tpu as pltpu

BLOCK_C = 2048


def _linear_block(x_ref, w_ref, b_ref, o_ref):
    acc = jax.lax.dot_general(
        x_ref[...],
        w_ref[...],
        (((1,), (1,)), ((), ())),
        preferred_element_type=jnp.float32,
    )
    o_ref[...] = acc + b_ref[...]


@jax.jit
def kernel(x, weight, bias):
    n_classes, _ = weight.shape
    batch = x.shape[0]
    bias2d = bias.reshape(1, n_classes)
    grid = (pl.cdiv(n_classes, BLOCK_C),)
    out = pl.pallas_call(
        _linear_block,
        grid=grid,
        in_specs=[
            pl.BlockSpec((batch, x.shape[1]), lambda i: (0, 0)),
            pl.BlockSpec((BLOCK_C, weight.shape[1]), lambda i: (i, 0)),
            pl.BlockSpec((1, BLOCK_C), lambda i: (0, i)),
        ],
        out_specs=pl.BlockSpec((batch, BLOCK_C), lambda i: (0, i)),
        out_shape=jax.ShapeDtypeStruct((batch, n_classes), jnp.float32),
        compiler_params=pltpu.CompilerParams(
            dimension_semantics=("parallel",),
        ),
    )(x, weight, bias2d)
    return out


# speedup vs baseline: 1.8146x; 1.8146x over previous
import functools

import jax
import jax.numpy as jnp
from jax.experimental import pallas as pl
from jax.experimental.pallas import tpu as pltpu

BLOCK_C = 8192


def _linear_block(x_ref, w_ref, b_ref, o_ref):
    acc = jax.lax.dot_general(
        x_ref[...],
        w_ref[...],
        (((1,), (1,)), ((), ())),
        preferred_element_type=jnp.float32,
    )
    o_ref[...] = acc + b_ref[...]


@jax.jit
def kernel(x, weight, bias):
    n_classes, _ = weight.shape
    batch = x.shape[0]
    bias2d = bias.reshape(1, n_classes)
    grid = (pl.cdiv(n_classes, BLOCK_C),)
    out = pl.pallas_call(
        _linear_block,
        grid=grid,
        in_specs=[
            pl.BlockSpec((batch, x.shape[1]), lambda i: (0, 0)),
            pl.BlockSpec((BLOCK_C, weight.shape[1]), lambda i: (i, 0)),
            pl.BlockSpec((1, BLOCK_C), lambda i: (0, i)),
        ],
        out_specs=pl.BlockSpec((batch, BLOCK_C), lambda i: (0, i)),
        out_shape=jax.ShapeDtypeStruct((batch, n_classes), jnp.float32),
        compiler_params=pltpu.CompilerParams(
            dimension_semantics=("parallel",),
        ),
    )(x, weight, bias2d)
    return out


# speedup vs baseline: 1.9578x; 1.0789x over previous
import functools

import jax
import jax.numpy as jnp
from jax.experimental import pallas as pl
from jax.experimental.pallas import tpu as pltpu

BLOCK_C = 16384


def _linear_block(x_ref, w_ref, b_ref, o_ref):
    acc = jax.lax.dot_general(
        x_ref[...],
        w_ref[...],
        (((1,), (1,)), ((), ())),
        preferred_element_type=jnp.float32,
    )
    o_ref[...] = acc + b_ref[...]


@jax.jit
def kernel(x, weight, bias):
    n_classes, _ = weight.shape
    batch = x.shape[0]
    bias2d = bias.reshape(1, n_classes)
    grid = (pl.cdiv(n_classes, BLOCK_C),)
    out = pl.pallas_call(
        _linear_block,
        grid=grid,
        in_specs=[
            pl.BlockSpec((batch, x.shape[1]), lambda i: (0, 0)),
            pl.BlockSpec((BLOCK_C, weight.shape[1]), lambda i: (i, 0)),
            pl.BlockSpec((1, BLOCK_C), lambda i: (0, i)),
        ],
        out_specs=pl.BlockSpec((batch, BLOCK_C), lambda i: (0, i)),
        out_shape=jax.ShapeDtypeStruct((batch, n_classes), jnp.float32),
        compiler_params=pltpu.CompilerParams(
            dimension_semantics=("parallel",),
        ),
    )(x, weight, bias2d)
    return out


# speedup vs baseline: 1.9601x; 1.0012x over previous
import functools

import jax
import jax.numpy as jnp
from jax.experimental import pallas as pl
from jax.experimental.pallas import tpu as pltpu

BLOCK_C = 20096


def _linear_block(x_ref, w_ref, b_ref, o_ref):
    acc = jax.lax.dot_general(
        x_ref[...],
        w_ref[...],
        (((1,), (1,)), ((), ())),
        preferred_element_type=jnp.float32,
    )
    o_ref[...] = acc + b_ref[...]


@jax.jit
def kernel(x, weight, bias):
    n_classes, _ = weight.shape
    batch = x.shape[0]
    bias2d = bias.reshape(1, n_classes)
    grid = (pl.cdiv(n_classes, BLOCK_C),)
    out = pl.pallas_call(
        _linear_block,
        grid=grid,
        in_specs=[
            pl.BlockSpec((batch, x.shape[1]), lambda i: (0, 0)),
            pl.BlockSpec((BLOCK_C, weight.shape[1]), lambda i: (i, 0)),
            pl.BlockSpec((1, BLOCK_C), lambda i: (0, i)),
        ],
        out_specs=pl.BlockSpec((batch, BLOCK_C), lambda i: (0, i)),
        out_shape=jax.ShapeDtypeStruct((batch, n_classes), jnp.float32),
        compiler_params=pltpu.CompilerParams(
            dimension_semantics=("parallel",),
        ),
    )(x, weight, bias2d)
    return out


# speedup vs baseline: 1.9623x; 1.0011x over previous
import jax
import jax.numpy as jnp
from jax.experimental import pallas as pl
from jax.experimental.pallas import tpu as pltpu

BLOCK_C = 10112
K_STREAMS = 2


def _linear_block(x_ref, *refs):
    w_refs = refs[:K_STREAMS]
    b_ref = refs[K_STREAMS]
    o_ref = refs[K_STREAMS + 1]
    for j in range(K_STREAMS):
        acc = jax.lax.dot_general(
            x_ref[...],
            w_refs[j][...],
            (((1,), (1,)), ((), ())),
            preferred_element_type=jnp.float32,
        )
        o_ref[:, j * BLOCK_C:(j + 1) * BLOCK_C] = (
            acc + b_ref[:, j * BLOCK_C:(j + 1) * BLOCK_C]
        )


@jax.jit
def kernel(x, weight, bias):
    n_classes, in_features = weight.shape
    batch = x.shape[0]
    bias2d = bias.reshape(1, n_classes)
    step_c = K_STREAMS * BLOCK_C
    grid = (pl.cdiv(n_classes, step_c),)

    def w_index(j):
        return lambda i: (i * K_STREAMS + j, 0)

    w_specs = [
        pl.BlockSpec((BLOCK_C, in_features), w_index(j)) for j in range(K_STREAMS)
    ]
    out = pl.pallas_call(
        _linear_block,
        grid=grid,
        in_specs=[
            pl.BlockSpec((batch, in_features), lambda i: (0, 0)),
            *w_specs,
            pl.BlockSpec((1, step_c), lambda i: (0, i)),
        ],
        out_specs=pl.BlockSpec((batch, step_c), lambda i: (0, i)),
        out_shape=jax.ShapeDtypeStruct((batch, n_classes), jnp.float32),
        compiler_params=pltpu.CompilerParams(
            dimension_semantics=("parallel",),
        ),
    )(x, *([weight] * K_STREAMS), bias2d)
    return out
